# Initial kernel scaffold; baseline (speedup 1.0000x reference)
import jax
import jax.numpy as jnp
from jax.experimental import pallas as pl


def kernel(tokens, table, prompt_embeds):
    raise NotImplementedError("write your pallas kernel here")


# speedup vs baseline: 5.7436x; 5.7436x over previous
import functools

import jax
import jax.numpy as jnp
from jax import lax
from jax.experimental import pallas as pl
from jax.experimental.pallas import tpu as pltpu
from jax.experimental.pallas import tpu_sc as plsc

VOCAB = 100000
H = 128
NP = 16
B = 1024
L = 200

NC = 2
NS = 16
NW = NC * NS
ROWS_PER_W = B // NW
CH = 100
NCH = L // CH
TCH = ROWS_PER_W * NCH
OUT_L = NP + L


def _soft_embedding_body(tokens_hbm, table_hbm, prompt_hbm, out_hbm,
                         idx_v, rows_v, gsem):
    wid = lax.axis_index("s") * NC + lax.axis_index("c")

    pltpu.sync_copy(tokens_hbm.at[pl.ds(wid * TCH, TCH)], idx_v)
    pltpu.sync_copy(prompt_hbm, rows_v.at[pl.ds(0, NP)])

    @pl.loop(0, ROWS_PER_W)
    def _row(i):
        b = wid * ROWS_PER_W + i
        out_base = b * OUT_L
        for j in range(NCH):
            c = i * NCH + j
            pltpu.async_copy(
                table_hbm.at[idx_v.at[c]],
                rows_v.at[pl.ds(NP + j * CH, CH)], gsem).wait()
        pltpu.sync_copy(rows_v, out_hbm.at[pl.ds(out_base, OUT_L)])


@jax.jit
def _soft_embedding(tokens2, table, prompt_embeds):
    mesh = plsc.VectorSubcoreMesh(
        core_axis_name="c", subcore_axis_name="s",
        num_cores=NC, num_subcores=NS)
    flat = pl.kernel(
        _soft_embedding_body,
        out_type=jax.ShapeDtypeStruct((B * OUT_L, H), jnp.float32),
        mesh=mesh,
        scratch_types=[
            pltpu.VMEM((TCH, CH), jnp.int32),
            pltpu.VMEM((OUT_L, H), jnp.float32),
            pltpu.SemaphoreType.DMA,
        ],
    )(tokens2, table, prompt_embeds)
    return flat.reshape(B, OUT_L, H)


def kernel(tokens, table, prompt_embeds):
    tokens2 = tokens.astype(jnp.int32).reshape(B * L // CH, CH)
    return _soft_embedding(tokens2, table, prompt_embeds)


# speedup vs baseline: 7.8507x; 1.3669x over previous
import functools

import jax
import jax.numpy as jnp
from jax import lax
from jax.experimental import pallas as pl
from jax.experimental.pallas import tpu as pltpu
from jax.experimental.pallas import tpu_sc as plsc

VOCAB = 100000
H = 128
NP = 16
B = 1024
L = 200

NC = 2
NS = 16
NW = NC * NS
ROWS_PER_W = B // NW
CH = 100
NCH = L // CH
TCH = ROWS_PER_W * NCH
OUT_L = NP + L
NBUF = 4


def _soft_embedding_body(tokens_hbm, table_hbm, prompt_hbm, out_hbm,
                         idx_v, rows_v, g0, g1, g2, g3, w0, w1, w2, w3):
    gsems = (g0, g1, g2, g3)
    wsems = (w0, w1, w2, w3)
    wid = lax.axis_index("s") * NC + lax.axis_index("c")
    first_row = wid * ROWS_PER_W

    pltpu.sync_copy(tokens_hbm.at[pl.ds(wid * TCH, TCH)], idx_v)
    for k in range(NBUF):
        pltpu.sync_copy(prompt_hbm, rows_v.at[k].at[pl.ds(0, NP)])

    def do_block(i0, first):
        descs = []
        for k in range(NBUF):
            i = i0 + k
            if not first:
                pltpu.make_async_copy(
                    rows_v.at[k], out_hbm.at[pl.ds(0, OUT_L)],
                    wsems[k]).wait()
            ds = []
            for j in range(NCH):
                ds.append(pltpu.async_copy(
                    table_hbm.at[idx_v.at[i * NCH + j]],
                    rows_v.at[k].at[pl.ds(NP + j * CH, CH)], gsems[k]))
            descs.append(ds)
        for k in range(NBUF):
            i = i0 + k
            for d in descs[k]:
                d.wait()
            pltpu.async_copy(
                rows_v.at[k],
                out_hbm.at[pl.ds((first_row + i) * OUT_L, OUT_L)], wsems[k])

    do_block(0, True)

    @pl.loop(NBUF, ROWS_PER_W, step=NBUF)
    def _block(i0):
        do_block(i0, False)

    for k in range(NBUF):
        pltpu.make_async_copy(
            rows_v.at[k], out_hbm.at[pl.ds(0, OUT_L)], wsems[k]).wait()


@jax.jit
def _soft_embedding(tokens2, table, prompt_embeds):
    mesh = plsc.VectorSubcoreMesh(
        core_axis_name="c", subcore_axis_name="s",
        num_cores=NC, num_subcores=NS)
    flat = pl.kernel(
        _soft_embedding_body,
        out_type=jax.ShapeDtypeStruct((B * OUT_L, H), jnp.float32),
        mesh=mesh,
        scratch_types=(
            [pltpu.VMEM((TCH, CH), jnp.int32),
             pltpu.VMEM((NBUF, OUT_L, H), jnp.float32)]
            + [pltpu.SemaphoreType.DMA] * (2 * NBUF)
        ),
    )(tokens2, table, prompt_embeds)
    return flat.reshape(B, OUT_L, H)


def kernel(tokens, table, prompt_embeds):
    tokens2 = tokens.astype(jnp.int32).reshape(B * L // CH, CH)
    return _soft_embedding(tokens2, table, prompt_embeds)


# speedup vs baseline: 8.2610x; 1.0523x over previous
import functools

import jax
import jax.numpy as jnp
from jax import lax
from jax.experimental import pallas as pl
from jax.experimental.pallas import tpu as pltpu
from jax.experimental.pallas import tpu_sc as plsc

VOCAB = 100000
H = 128
NP = 16
B = 1024
L = 200

NC = 2
NS = 16
NW = NC * NS
ROWS_PER_W = B // NW
CH = 40
NCH = L // CH
TCH = ROWS_PER_W * NCH
OUT_L = NP + L
NBUF = 8


def _soft_embedding_body(tokens_hbm, table_hbm, prompt_hbm, out_hbm,
                         idx_v, rows_v, prompt_v, *sems):
    gsems = sems[:NBUF]
    wsems = sems[NBUF:2 * NBUF]
    psem = sems[2 * NBUF]
    wid = lax.axis_index("s") * NC + lax.axis_index("c")
    first_row = wid * ROWS_PER_W

    pltpu.sync_copy(tokens_hbm.at[pl.ds(wid * TCH, TCH)], idx_v)
    pltpu.sync_copy(prompt_hbm, prompt_v)

    def do_block(c0, first):
        descs = []
        for k in range(NBUF):
            c = c0 + k
            if not first:
                pltpu.make_async_copy(
                    rows_v.at[k], out_hbm.at[pl.ds(0, CH)], wsems[k]).wait()
            descs.append(pltpu.async_copy(
                table_hbm.at[idx_v.at[c]], rows_v.at[k], gsems[k]))

            @pl.when(c % NCH == 0)
            def _():
                pltpu.async_copy(
                    prompt_v,
                    out_hbm.at[pl.ds((first_row + c // NCH) * OUT_L, NP)],
                    psem)

        for k in range(NBUF):
            c = c0 + k
            descs[k].wait()
            b = first_row + c // NCH
            base = b * OUT_L + NP + (c % NCH) * CH
            pltpu.async_copy(
                rows_v.at[k], out_hbm.at[pl.ds(base, CH)], wsems[k])

    do_block(0, True)

    @pl.loop(NBUF, TCH, step=NBUF)
    def _block(c0):
        do_block(c0, False)

    for k in range(NBUF):
        pltpu.make_async_copy(
            rows_v.at[k], out_hbm.at[pl.ds(0, CH)], wsems[k]).wait()

    @pl.loop(0, ROWS_PER_W)
    def _drain_prompt(i):
        pltpu.make_async_copy(
            prompt_v, out_hbm.at[pl.ds(0, NP)], psem).wait()


@jax.jit
def _soft_embedding(tokens2, table, prompt_embeds):
    mesh = plsc.VectorSubcoreMesh(
        core_axis_name="c", subcore_axis_name="s",
        num_cores=NC, num_subcores=NS)
    flat = pl.kernel(
        _soft_embedding_body,
        out_type=jax.ShapeDtypeStruct((B * OUT_L, H), jnp.float32),
        mesh=mesh,
        scratch_types=(
            [pltpu.VMEM((TCH, CH), jnp.int32),
             pltpu.VMEM((NBUF, CH, H), jnp.float32),
             pltpu.VMEM((NP, H), jnp.float32)]
            + [pltpu.SemaphoreType.DMA] * (2 * NBUF + 1)
        ),
    )(tokens2, table, prompt_embeds)
    return flat.reshape(B, OUT_L, H)


def kernel(tokens, table, prompt_embeds):
    tokens2 = tokens.astype(jnp.int32).reshape(B * L // CH, CH)
    return _soft_embedding(tokens2, table, prompt_embeds)
